# Initial kernel scaffold; baseline (speedup 1.0000x reference)
import functools

import jax
import jax.numpy as jnp
from jax import lax
from jax.experimental import pallas as pl
from jax.experimental.pallas import tpu as pltpu
from jax.experimental.pallas import tpu_sc as plsc

N = 10000
E = 320000
D = 128
NC = 2
NS = 16
L = 16
EPT = E // (NC * NS)
B = 80
RPT = N // NS
ZR = 125

_f32 = jnp.float32


def _proj_tail(h, asv_ref, adv_ref, h_ref, as_ref, ad_ref, c_ref):
    h_ref[...] = h
    a_s = jnp.sum(h * asv_ref[...], axis=1, keepdims=True)
    a_d = jnp.sum(h * adv_ref[...], axis=1, keepdims=True)
    as_ref[...] = a_s
    ad_ref[...] = a_d
    m = jnp.max(a_s) + jnp.max(a_d)
    c_ref[0, 0] = jnp.where(m >= 0.0, m, 0.2 * m)


def _mat1_body(x_ref, w_ref, asv_ref, adv_ref, h_ref, as_ref, ad_ref, c_ref):
    h = jnp.dot(x_ref[...], w_ref[...], precision=lax.Precision.HIGHEST,
                preferred_element_type=_f32)
    _proj_tail(h, asv_ref, adv_ref, h_ref, as_ref, ad_ref, c_ref)


def _combine_mat_body(u0_ref, u1_ref, d0_ref, d1_ref, bias_ref, gamma_ref,
                      beta_ref, w_ref, asv_ref, adv_ref,
                      h_ref, as_ref, ad_ref, c_ref):
    denom = d0_ref[...] + d1_ref[...] + 1e-16
    z = (u0_ref[...] + u1_ref[...]) / denom + bias_ref[...]
    mean = jnp.mean(z, axis=0, keepdims=True)
    var = jnp.mean((z - mean) ** 2, axis=0, keepdims=True)
    z = gamma_ref[...] * (z - mean) / jnp.sqrt(var + 1e-5) + beta_ref[...]
    z = jnp.maximum(z, 0.0)
    h = jnp.dot(z, w_ref[...], precision=lax.Precision.HIGHEST,
                preferred_element_type=_f32)
    _proj_tail(h, asv_ref, adv_ref, h_ref, as_ref, ad_ref, c_ref)


def _final_body(u0_ref, u1_ref, d0_ref, d1_ref, bias_ref, o_ref):
    denom = d0_ref[...] + d1_ref[...] + 1e-16
    z = (u0_ref[...] + u1_ref[...]) / denom + bias_ref[...]
    o_ref[...] = jnp.maximum(z, 0.0)


_mat1 = pl.pallas_call(
    _mat1_body,
    out_shape=(jax.ShapeDtypeStruct((N, D), _f32),
               jax.ShapeDtypeStruct((N, 1), _f32),
               jax.ShapeDtypeStruct((N, 1), _f32),
               jax.ShapeDtypeStruct((1, 1), _f32)))

_combine_mat = pl.pallas_call(
    _combine_mat_body,
    out_shape=(jax.ShapeDtypeStruct((N, D), _f32),
               jax.ShapeDtypeStruct((N, 1), _f32),
               jax.ShapeDtypeStruct((N, 1), _f32),
               jax.ShapeDtypeStruct((1, 1), _f32)))

_final = pl.pallas_call(
    _final_body,
    out_shape=jax.ShapeDtypeStruct((N, D), _f32))


def _edge_body(h_hbm, as_hbm, ad_hbm, src_hbm, dst_hbm, c_hbm,
               u_hbm, dn_hbm,
               asv, adv, cv, srcv, dstv, rowsv, pv, pcolv, zuv, zdv,
               u_sh, dn_sh, sem):
    c_idx = lax.axis_index("c")
    s_idx = lax.axis_index("s")
    wid = c_idx * NS + s_idx

    zf = jnp.zeros((L,), _f32)

    @pl.loop(0, ZR)
    def _zu(j):
        for k in range(D // L):
            zuv[j, pl.ds(k * L, L)] = zf

    @pl.loop(0, RPT)
    def _zd(j):
        zdv[j] = zf

    @pl.loop(0, RPT // ZR)
    def _zs(i):
        pltpu.sync_copy(zuv, u_sh.at[pl.ds(s_idx * RPT + i * ZR, ZR)])

    pltpu.sync_copy(zdv, dn_sh.at[pl.ds(s_idx * RPT, RPT)])

    pltpu.sync_copy(as_hbm, asv)
    pltpu.sync_copy(ad_hbm, adv)
    pltpu.sync_copy(c_hbm, cv)

    plsc.subcore_barrier()

    cvec = cv[...]
    col0 = jnp.zeros((L,), jnp.int32)

    ebase = wid * EPT

    @pl.loop(0, EPT // B)
    def _chunk(ci):
        base = ebase + ci * B
        pltpu.sync_copy(src_hbm.at[pl.ds(base, B)], srcv)
        pltpu.sync_copy(dst_hbm.at[pl.ds(base, B)], dstv)
        gat = pltpu.async_copy(h_hbm.at[srcv], rowsv, sem)
        for g in range(B // L):
            s16 = srcv[pl.ds(g * L, L)]
            d16 = dstv[pl.ds(g * L, L)]
            a = plsc.load_gather(asv, [s16]) + plsc.load_gather(adv, [d16])
            e = jnp.where(a >= 0.0, a, 0.2 * a)
            p = jnp.exp(e - cvec)
            pv[pl.ds(g * L, L)] = p
            rid = g * L + lax.iota(jnp.int32, L)
            plsc.store_scatter(pcolv, [rid, col0], p)
        gat.wait()

        @pl.loop(0, B)
        def _scale(j):
            pj = plsc.load_gather(pv, [jnp.broadcast_to(j, (L,))])
            for k in range(D // L):
                sl = pl.ds(k * L, L)
                rowsv[j, sl] = rowsv[j, sl] * pj

        pltpu.sync_copy(rowsv, u_sh.at[dstv], add=True)
        pltpu.sync_copy(pcolv, dn_sh.at[dstv], add=True)

    plsc.subcore_barrier()

    @pl.loop(0, RPT // ZR)
    def _ou(i):
        r = s_idx * RPT + i * ZR
        pltpu.sync_copy(u_sh.at[pl.ds(r, ZR)], u_hbm.at[c_idx, pl.ds(r, ZR)])

    r0 = s_idx * RPT
    pltpu.sync_copy(dn_sh.at[pl.ds(r0, RPT)], dn_hbm.at[c_idx, pl.ds(r0, RPT)])


_edge_sc = pl.kernel(
    _edge_body,
    out_type=(jax.ShapeDtypeStruct((NC, N, D), _f32),
              jax.ShapeDtypeStruct((NC, N, 16), _f32)),
    mesh=plsc.VectorSubcoreMesh(core_axis_name="c", subcore_axis_name="s"),
    scratch_types=[
        pltpu.VMEM((N,), _f32),
        pltpu.VMEM((N,), _f32),
        pltpu.VMEM((L,), _f32),
        pltpu.VMEM((B,), jnp.int32),
        pltpu.VMEM((B,), jnp.int32),
        pltpu.VMEM((B, D), _f32),
        pltpu.VMEM((B,), _f32),
        pltpu.VMEM((B, 16), _f32),
        pltpu.VMEM((ZR, D), _f32),
        pltpu.VMEM((RPT, 16), _f32),
        pltpu.VMEM_SHARED((N, D), _f32),
        pltpu.VMEM_SHARED((N, 16), _f32),
        pltpu.SemaphoreType.DMA,
    ])


def _layer_edges(h, a_s, a_d, src, dst, c):
    c16 = jnp.broadcast_to(jnp.reshape(c, ()), (16,))
    u, dn = _edge_sc(h, a_s.reshape(N), a_d.reshape(N), src, dst, c16)
    return u[0], u[1], dn[0, :, 0:1], dn[1, :, 0:1]


def kernel(x, edge_index, W1, att_src1, att_dst1, bias1, gamma1, beta1,
           W2, att_src2, att_dst2, bias2):
    src = edge_index[0]
    dst = edge_index[1]
    h1, as1, ad1, c1 = _mat1(x, W1, att_src1.reshape(1, D),
                             att_dst1.reshape(1, D))
    u0, u1, d0, d1 = _layer_edges(h1, as1, ad1, src, dst, c1)
    h2, as2, ad2, c2 = _combine_mat(u0, u1, d0, d1, bias1.reshape(1, D),
                                    gamma1.reshape(1, D), beta1.reshape(1, D),
                                    W2, att_src2.reshape(1, D),
                                    att_dst2.reshape(1, D))
    u0, u1, d0, d1 = _layer_edges(h2, as2, ad2, src, dst, c2)
    return _final(u0, u1, d0, d1, bias2.reshape(1, D))


# speedup vs baseline: 21.5774x; 21.5774x over previous
import dataclasses
import functools

import jax
import jax.numpy as jnp
from jax import lax
from jax.experimental import pallas as pl
from jax.experimental.pallas import tpu as pltpu
from jax.experimental.pallas import tpu_sc as plsc

N = 10000
E = 320000
D = 128
NC = 2
NS = 16
L = 16
EPT = E // (NC * NS)
B = 80
NPAD = 10240
RPT = NPAD // NS
ZR = 128

_f32 = jnp.float32


def _proj_tail(h, asv_ref, adv_ref, h_ref, as_ref, ad_ref, c_ref):
    h_ref[...] = h
    a_s = jnp.sum(h * asv_ref[...], axis=1, keepdims=True)
    a_d = jnp.sum(h * adv_ref[...], axis=1, keepdims=True)
    as_ref[...] = a_s
    ad_ref[...] = a_d
    m = jnp.max(a_s) + jnp.max(a_d)
    c_ref[0, 0] = jnp.where(m >= 0.0, m, 0.2 * m)


def _mat1_body(x_ref, w_ref, asv_ref, adv_ref, h_ref, as_ref, ad_ref, c_ref):
    h = jnp.dot(x_ref[...], w_ref[...], precision=lax.Precision.HIGHEST,
                preferred_element_type=_f32)
    _proj_tail(h, asv_ref, adv_ref, h_ref, as_ref, ad_ref, c_ref)


def _combine_mat_body(u0_ref, u1_ref, d0_ref, d1_ref, bias_ref, gamma_ref,
                      beta_ref, w_ref, asv_ref, adv_ref,
                      h_ref, as_ref, ad_ref, c_ref):
    denom = d0_ref[...] + d1_ref[...] + 1e-16
    z = (u0_ref[...] + u1_ref[...]) / denom + bias_ref[...]
    mean = jnp.mean(z, axis=0, keepdims=True)
    var = jnp.mean((z - mean) ** 2, axis=0, keepdims=True)
    z = gamma_ref[...] * (z - mean) / jnp.sqrt(var + 1e-5) + beta_ref[...]
    z = jnp.maximum(z, 0.0)
    h = jnp.dot(z, w_ref[...], precision=lax.Precision.HIGHEST,
                preferred_element_type=_f32)
    _proj_tail(h, asv_ref, adv_ref, h_ref, as_ref, ad_ref, c_ref)


def _final_body(u0_ref, u1_ref, d0_ref, d1_ref, bias_ref, o_ref):
    denom = d0_ref[...] + d1_ref[...] + 1e-16
    z = (u0_ref[...] + u1_ref[...]) / denom + bias_ref[...]
    o_ref[...] = jnp.maximum(z, 0.0)


_proj_out_shape = (jax.ShapeDtypeStruct((N, D), _f32),
                   jax.ShapeDtypeStruct((N, 1), _f32),
                   jax.ShapeDtypeStruct((N, 1), _f32),
                   jax.ShapeDtypeStruct((1, 1), _f32))
_proj_out_specs = (pl.BlockSpec(memory_space=pltpu.VMEM),
                   pl.BlockSpec(memory_space=pltpu.VMEM),
                   pl.BlockSpec(memory_space=pltpu.VMEM),
                   pl.BlockSpec(memory_space=pltpu.SMEM))

_mat1 = pl.pallas_call(
    _mat1_body, out_shape=_proj_out_shape, out_specs=_proj_out_specs)

_combine_mat = pl.pallas_call(
    _combine_mat_body, out_shape=_proj_out_shape, out_specs=_proj_out_specs)

_final = pl.pallas_call(
    _final_body,
    out_shape=jax.ShapeDtypeStruct((N, D), _f32))


def _edge_body(h_hbm, as_hbm, ad_hbm, src_hbm, dst_hbm, c_hbm,
               u_hbm, dn_hbm,
               asv, adv, cv, srcv, dstv, rowsv, pv, zuv, zdv,
               u_sh, dn_sh, sem):
    c_idx = lax.axis_index("c")
    s_idx = lax.axis_index("s")
    wid = c_idx * NS + s_idx

    zf = jnp.zeros((L,), _f32)

    @pl.loop(0, ZR)
    def _zu(j):
        for k in range(D // L):
            zuv[j, pl.ds(k * L, L)] = zf

    @pl.loop(0, RPT // L)
    def _zd(j):
        zdv[pl.ds(j * L, L)] = zf

    @pl.loop(0, RPT // ZR)
    def _zs(i):
        pltpu.sync_copy(zuv, u_sh.at[pl.ds(s_idx * RPT + i * ZR, ZR)])

    pltpu.sync_copy(zdv, dn_sh.at[pl.ds(s_idx * RPT, RPT)])

    pltpu.sync_copy(as_hbm, asv)
    pltpu.sync_copy(ad_hbm, adv)
    pltpu.sync_copy(c_hbm, cv)

    plsc.subcore_barrier()

    cvec = cv[...]
    ebase = wid * EPT

    @pl.loop(0, EPT // B)
    def _chunk(ci):
        base = ebase + ci * B
        pltpu.sync_copy(src_hbm.at[pl.ds(base, B)], srcv)
        pltpu.sync_copy(dst_hbm.at[pl.ds(base, B)], dstv)
        gat = pltpu.async_copy(h_hbm.at[srcv], rowsv, sem)
        for g in range(B // L):
            s16 = srcv[pl.ds(g * L, L)]
            d16 = dstv[pl.ds(g * L, L)]
            a = plsc.load_gather(asv, [s16]) + plsc.load_gather(adv, [d16])
            e = jnp.where(a >= 0.0, a, 0.2 * a)
            p = jnp.exp(e - cvec)
            pv[pl.ds(g * L, L)] = p
        gat.wait()

        @pl.loop(0, B)
        def _scale(j):
            pj = plsc.load_gather(pv, [jnp.broadcast_to(j, (L,))])
            for k in range(D // L):
                sl = pl.ds(k * L, L)
                rowsv[j, sl] = rowsv[j, sl] * pj

        pltpu.sync_copy(rowsv, u_sh.at[dstv], add=True)
        pltpu.sync_copy(pv, dn_sh.at[dstv], add=True)

    plsc.subcore_barrier()

    r0 = s_idx * RPT
    pltpu.sync_copy(u_sh.at[pl.ds(r0, RPT)], u_hbm.at[c_idx, pl.ds(r0, RPT)])
    pltpu.sync_copy(dn_sh.at[pl.ds(r0, RPT)], dn_hbm.at[c_idx, pl.ds(r0, RPT)])


def _sc_compiler_params():
    cp = pltpu.CompilerParams()
    fields = pltpu.CompilerParams.__dataclass_fields__
    if "needs_layout_passes" in fields:
        cp = dataclasses.replace(cp, needs_layout_passes=False)
    if "use_tc_tiling_on_sc" in fields:
        cp = dataclasses.replace(cp, use_tc_tiling_on_sc=False)
    return cp


@functools.cache
def _edge_sc_kernel():
  return pl.kernel(
    _edge_body,
    compiler_params=_sc_compiler_params(),
    out_type=(jax.ShapeDtypeStruct((NC, NPAD, D), _f32),
              jax.ShapeDtypeStruct((NC, NPAD), _f32)),
    mesh=plsc.VectorSubcoreMesh(core_axis_name="c", subcore_axis_name="s",
                                num_cores=NC, num_subcores=NS),
    scratch_types=[
        pltpu.VMEM((N,), _f32),
        pltpu.VMEM((N,), _f32),
        pltpu.VMEM((L,), _f32),
        pltpu.VMEM((B,), jnp.int32),
        pltpu.VMEM((B,), jnp.int32),
        pltpu.VMEM((B, D), _f32),
        pltpu.VMEM((B,), _f32),
        pltpu.VMEM((ZR, D), _f32),
        pltpu.VMEM((RPT,), _f32),
        pltpu.VMEM_SHARED((NPAD, D), _f32),
        pltpu.VMEM_SHARED((NPAD,), _f32),
        pltpu.SemaphoreType.DMA,
    ])


def _layer_edges(h, a_s, a_d, src, dst, c):
    c16 = jnp.broadcast_to(jnp.reshape(c, ()), (16,))
    u, dn = _edge_sc_kernel()(h, a_s.reshape(N), a_d.reshape(N), src, dst, c16)
    return (u[0, :N], u[1, :N],
            dn[0, :N].reshape(N, 1), dn[1, :N].reshape(N, 1))


def kernel(x, edge_index, W1, att_src1, att_dst1, bias1, gamma1, beta1,
           W2, att_src2, att_dst2, bias2):
    src = edge_index[0]
    dst = edge_index[1]
    h1, as1, ad1, c1 = _mat1(x, W1, att_src1.reshape(1, D),
                             att_dst1.reshape(1, D))
    u0, u1, d0, d1 = _layer_edges(h1, as1, ad1, src, dst, c1)
    h2, as2, ad2, c2 = _combine_mat(u0, u1, d0, d1, bias1.reshape(1, D),
                                    gamma1.reshape(1, D), beta1.reshape(1, D),
                                    W2, att_src2.reshape(1, D),
                                    att_dst2.reshape(1, D))
    u0, u1, d0, d1 = _layer_edges(h2, as2, ad2, src, dst, c2)
    return _final(u0, u1, d0, d1, bias2.reshape(1, D))


# speedup vs baseline: 28.5393x; 1.3226x over previous
import dataclasses
import functools

import jax
import jax.numpy as jnp
from jax import lax
from jax.experimental import pallas as pl
from jax.experimental.pallas import tpu as pltpu
from jax.experimental.pallas import tpu_sc as plsc

N = 10000
E = 320000
D = 128
NC = 2
NS = 16
L = 16
EPT = E // (NC * NS)
B = 80
NB = 2
NPAD = 10240
RPT = NPAD // NS

_f32 = jnp.float32


def _proj_tail(h, asv_ref, adv_ref, h_ref, as_ref, ad_ref, c_ref):
    h_ref[...] = h
    a_s = jnp.sum(h * asv_ref[...], axis=1, keepdims=True)
    a_d = jnp.sum(h * adv_ref[...], axis=1, keepdims=True)
    as_ref[...] = a_s
    ad_ref[...] = a_d
    m = jnp.max(a_s) + jnp.max(a_d)
    c_ref[0, 0] = jnp.where(m >= 0.0, m, 0.2 * m)


def _mat1_body(x_ref, w_ref, asv_ref, adv_ref, h_ref, as_ref, ad_ref, c_ref):
    h = jnp.dot(x_ref[...], w_ref[...], precision=lax.Precision.HIGHEST,
                preferred_element_type=_f32)
    _proj_tail(h, asv_ref, adv_ref, h_ref, as_ref, ad_ref, c_ref)


def _combine_mat_body(u0_ref, u1_ref, d0_ref, d1_ref, bias_ref, gamma_ref,
                      beta_ref, w_ref, asv_ref, adv_ref,
                      h_ref, as_ref, ad_ref, c_ref):
    denom = d0_ref[...] + d1_ref[...] + 1e-16
    z = (u0_ref[...] + u1_ref[...]) / denom + bias_ref[...]
    mean = jnp.mean(z, axis=0, keepdims=True)
    var = jnp.mean((z - mean) ** 2, axis=0, keepdims=True)
    z = gamma_ref[...] * (z - mean) / jnp.sqrt(var + 1e-5) + beta_ref[...]
    z = jnp.maximum(z, 0.0)
    h = jnp.dot(z, w_ref[...], precision=lax.Precision.HIGHEST,
                preferred_element_type=_f32)
    _proj_tail(h, asv_ref, adv_ref, h_ref, as_ref, ad_ref, c_ref)


def _final_body(u0_ref, u1_ref, d0_ref, d1_ref, bias_ref, o_ref):
    denom = d0_ref[...] + d1_ref[...] + 1e-16
    z = (u0_ref[...] + u1_ref[...]) / denom + bias_ref[...]
    o_ref[...] = jnp.maximum(z, 0.0)


_proj_out_shape = (jax.ShapeDtypeStruct((N, D), _f32),
                   jax.ShapeDtypeStruct((N, 1), _f32),
                   jax.ShapeDtypeStruct((N, 1), _f32),
                   jax.ShapeDtypeStruct((1, 1), _f32))
_proj_out_specs = (pl.BlockSpec(memory_space=pltpu.VMEM),
                   pl.BlockSpec(memory_space=pltpu.VMEM),
                   pl.BlockSpec(memory_space=pltpu.VMEM),
                   pl.BlockSpec(memory_space=pltpu.SMEM))

_mat1 = pl.pallas_call(
    _mat1_body, out_shape=_proj_out_shape, out_specs=_proj_out_specs)

_combine_mat = pl.pallas_call(
    _combine_mat_body, out_shape=_proj_out_shape, out_specs=_proj_out_specs)

_final = pl.pallas_call(
    _final_body,
    out_shape=jax.ShapeDtypeStruct((N, D), _f32))


def _edge_body(h_hbm, as_hbm, ad_hbm, src_hbm, dst_hbm, c_hbm,
               u_hbm, dn_hbm,
               asv, adv, cv,
               srcv0, srcv1, dstv0, dstv1,
               rowsv0, rowsv1, pv0, pv1, zdv,
               u_sh, dn_sh, gsem, usem, dsem):
    srcvs = (srcv0, srcv1)
    dstvs = (dstv0, dstv1)
    rowsvs = (rowsv0, rowsv1)
    pvs = (pv0, pv1)
    c_idx = lax.axis_index("c")
    s_idx = lax.axis_index("s")
    wid = c_idx * NS + s_idx

    zf = jnp.zeros((L,), _f32)

    @pl.loop(0, B)
    def _zu(j):
        for k in range(D // L):
            rowsv0[j, pl.ds(k * L, L)] = zf

    @pl.loop(0, RPT // L)
    def _zd(j):
        zdv[pl.ds(j * L, L)] = zf

    @pl.loop(0, RPT // B)
    def _zs(i):
        pltpu.sync_copy(rowsv0, u_sh.at[pl.ds(s_idx * RPT + i * B, B)])

    pltpu.sync_copy(zdv, dn_sh.at[pl.ds(s_idx * RPT, RPT)])

    pltpu.sync_copy(as_hbm, asv)
    pltpu.sync_copy(ad_hbm, adv)
    pltpu.sync_copy(c_hbm, cv)

    plsc.subcore_barrier()

    cvec = cv[...]
    ebase = wid * EPT
    NCH = EPT // B

    def _issue(c, b):
        base = ebase + c * B
        pltpu.sync_copy(src_hbm.at[pl.ds(base, B)], srcvs[b])
        pltpu.sync_copy(dst_hbm.at[pl.ds(base, B)], dstvs[b])
        pltpu.async_copy(h_hbm.at[srcvs[b]], rowsvs[b], gsem.at[b])

    def _wait_gather(b):
        pltpu.make_async_copy(h_hbm.at[srcvs[b]], rowsvs[b],
                              gsem.at[b]).wait()

    def _wait_scatters(b):
        pltpu.make_async_copy(rowsvs[b], u_sh.at[dstvs[b]],
                              usem.at[b]).wait()
        pltpu.make_async_copy(pvs[b], dn_sh.at[dstvs[b]],
                              dsem.at[b]).wait()

    def _compute(c, b):
        for g in range(B // L):
            s16 = srcvs[b][pl.ds(g * L, L)]
            d16 = dstvs[b][pl.ds(g * L, L)]
            a = plsc.load_gather(asv, [s16]) + plsc.load_gather(adv, [d16])
            e = jnp.where(a >= 0.0, a, 0.2 * a)
            p = jnp.exp(e - cvec)
            pvs[b][pl.ds(g * L, L)] = p

        @pl.loop(0, B)
        def _scale(j):
            pj = plsc.load_gather(pvs[b], [jnp.broadcast_to(j, (L,))])
            for k in range(D // L):
                sl = pl.ds(k * L, L)
                rowsvs[b][j, sl] = rowsvs[b][j, sl] * pj

        pltpu.sync_copy(rowsvs[b], u_sh.at[dstvs[b]], add=True)
        pltpu.sync_copy(pvs[b], dn_sh.at[dstvs[b]], add=True)

    _issue(0, 0)

    @pl.loop(0, (NCH - 1) // NB)
    def _ring(k):
        for b in range(NB):
            c = k * NB + b
            nb = (b + 1) % NB
            _wait_gather(b)
            base = ebase + (c + 1) * B
            pltpu.sync_copy(src_hbm.at[pl.ds(base, B)], srcvs[nb])
            pltpu.sync_copy(dst_hbm.at[pl.ds(base, B)], dstvs[nb])
            pltpu.async_copy(h_hbm.at[srcvs[nb]], rowsvs[nb], gsem.at[nb])
            _compute(c, b)

    _wait_gather(0)
    _compute(NCH - 1, 0)

    plsc.subcore_barrier()

    r0 = s_idx * RPT
    pltpu.sync_copy(u_sh.at[pl.ds(r0, RPT)], u_hbm.at[c_idx, pl.ds(r0, RPT)])
    pltpu.sync_copy(dn_sh.at[pl.ds(r0, RPT)], dn_hbm.at[c_idx, pl.ds(r0, RPT)])


def _sc_compiler_params():
    cp = pltpu.CompilerParams()
    fields = pltpu.CompilerParams.__dataclass_fields__
    if "needs_layout_passes" in fields:
        cp = dataclasses.replace(cp, needs_layout_passes=False)
    if "use_tc_tiling_on_sc" in fields:
        cp = dataclasses.replace(cp, use_tc_tiling_on_sc=False)
    return cp


@functools.cache
def _edge_sc_kernel():
  return pl.kernel(
    _edge_body,
    compiler_params=_sc_compiler_params(),
    out_type=(jax.ShapeDtypeStruct((NC, NPAD, D), _f32),
              jax.ShapeDtypeStruct((NC, NPAD), _f32)),
    mesh=plsc.VectorSubcoreMesh(core_axis_name="c", subcore_axis_name="s",
                                num_cores=NC, num_subcores=NS),
    scratch_types=[
        pltpu.VMEM((N,), _f32),
        pltpu.VMEM((N,), _f32),
        pltpu.VMEM((L,), _f32),
        pltpu.VMEM((B,), jnp.int32),
        pltpu.VMEM((B,), jnp.int32),
        pltpu.VMEM((B,), jnp.int32),
        pltpu.VMEM((B,), jnp.int32),
        pltpu.VMEM((B, D), _f32),
        pltpu.VMEM((B, D), _f32),
        pltpu.VMEM((B,), _f32),
        pltpu.VMEM((B,), _f32),
        pltpu.VMEM((RPT,), _f32),
        pltpu.VMEM_SHARED((NPAD, D), _f32),
        pltpu.VMEM_SHARED((NPAD,), _f32),
        pltpu.SemaphoreType.DMA((NB,)),
        pltpu.SemaphoreType.DMA((NB,)),
        pltpu.SemaphoreType.DMA((NB,)),
    ])


def _layer_edges(h, a_s, a_d, src, dst, c):
    c16 = jnp.broadcast_to(jnp.reshape(c, ()), (16,))
    u, dn = _edge_sc_kernel()(h, a_s.reshape(N), a_d.reshape(N), src, dst, c16)
    return (u[0, :N], u[1, :N],
            dn[0, :N].reshape(N, 1), dn[1, :N].reshape(N, 1))


def kernel(x, edge_index, W1, att_src1, att_dst1, bias1, gamma1, beta1,
           W2, att_src2, att_dst2, bias2):
    src = edge_index[0]
    dst = edge_index[1]
    h1, as1, ad1, c1 = _mat1(x, W1, att_src1.reshape(1, D),
                             att_dst1.reshape(1, D))
    u0, u1, d0, d1 = _layer_edges(h1, as1, ad1, src, dst, c1)
    h2, as2, ad2, c2 = _combine_mat(u0, u1, d0, d1, bias1.reshape(1, D),
                                    gamma1.reshape(1, D), beta1.reshape(1, D),
                                    W2, att_src2.reshape(1, D),
                                    att_dst2.reshape(1, D))
    u0, u1, d0, d1 = _layer_edges(h2, as2, ad2, src, dst, c2)
    return _final(u0, u1, d0, d1, bias2.reshape(1, D))


# speedup vs baseline: 32.0868x; 1.1243x over previous
import dataclasses
import functools

import jax
import jax.numpy as jnp
from jax import lax
from jax.experimental import pallas as pl
from jax.experimental.pallas import tpu as pltpu
from jax.experimental.pallas import tpu_sc as plsc

N = 10000
E = 320000
D = 128
NC = 2
NS = 16
L = 16
EPT = E // (NC * NS)
B = 80
NB = 3
NPAD = 10240
RPT = NPAD // NS

_f32 = jnp.float32


def _proj_tail(h, asv_ref, adv_ref, h_ref, as_ref, ad_ref, c_ref):
    h_ref[...] = h
    a_s = jnp.sum(h * asv_ref[...], axis=1, keepdims=True)
    a_d = jnp.sum(h * adv_ref[...], axis=1, keepdims=True)
    as_ref[...] = a_s
    ad_ref[...] = a_d
    m = jnp.max(a_s) + jnp.max(a_d)
    c_ref[0, 0] = jnp.where(m >= 0.0, m, 0.2 * m)


def _mat1_body(x_ref, w_ref, asv_ref, adv_ref, h_ref, as_ref, ad_ref, c_ref):
    h = jnp.dot(x_ref[...], w_ref[...], precision=lax.Precision.HIGHEST,
                preferred_element_type=_f32)
    _proj_tail(h, asv_ref, adv_ref, h_ref, as_ref, ad_ref, c_ref)


def _combine_mat_body(u0_ref, u1_ref, d0_ref, d1_ref, bias_ref, gamma_ref,
                      beta_ref, w_ref, asv_ref, adv_ref,
                      h_ref, as_ref, ad_ref, c_ref):
    denom = d0_ref[...] + d1_ref[...] + 1e-16
    z = (u0_ref[...] + u1_ref[...]) / denom + bias_ref[...]
    mean = jnp.mean(z, axis=0, keepdims=True)
    var = jnp.mean((z - mean) ** 2, axis=0, keepdims=True)
    z = gamma_ref[...] * (z - mean) / jnp.sqrt(var + 1e-5) + beta_ref[...]
    z = jnp.maximum(z, 0.0)
    h = jnp.dot(z, w_ref[...], precision=lax.Precision.HIGHEST,
                preferred_element_type=_f32)
    _proj_tail(h, asv_ref, adv_ref, h_ref, as_ref, ad_ref, c_ref)


def _final_body(u0_ref, u1_ref, d0_ref, d1_ref, bias_ref, o_ref):
    denom = d0_ref[...] + d1_ref[...] + 1e-16
    z = (u0_ref[...] + u1_ref[...]) / denom + bias_ref[...]
    o_ref[...] = jnp.maximum(z, 0.0)


_proj_out_shape = (jax.ShapeDtypeStruct((N, D), _f32),
                   jax.ShapeDtypeStruct((N, 1), _f32),
                   jax.ShapeDtypeStruct((N, 1), _f32),
                   jax.ShapeDtypeStruct((1, 1), _f32))
_proj_out_specs = (pl.BlockSpec(memory_space=pltpu.VMEM),
                   pl.BlockSpec(memory_space=pltpu.VMEM),
                   pl.BlockSpec(memory_space=pltpu.VMEM),
                   pl.BlockSpec(memory_space=pltpu.SMEM))

_mat1 = pl.pallas_call(
    _mat1_body, out_shape=_proj_out_shape, out_specs=_proj_out_specs)

_combine_mat = pl.pallas_call(
    _combine_mat_body, out_shape=_proj_out_shape, out_specs=_proj_out_specs)

_final = pl.pallas_call(
    _final_body,
    out_shape=jax.ShapeDtypeStruct((N, D), _f32))


def _edge_body(h_hbm, as_hbm, ad_hbm, src_hbm, dst_hbm, c_hbm,
               u_hbm, dn_hbm,
               asv, cv,
               srcv0, srcv1, srcv2, dstv0, dstv1, dstv2,
               rowsv0, rowsv1, rowsv2, pv0, pv1, pv2,
               adg0, adg1, adg2, zdv,
               u_sh, dn_sh, gsem, g2sem, usem, dsem):
    srcvs = (srcv0, srcv1, srcv2)
    dstvs = (dstv0, dstv1, dstv2)
    rowsvs = (rowsv0, rowsv1, rowsv2)
    pvs = (pv0, pv1, pv2)
    adgs = (adg0, adg1, adg2)
    c_idx = lax.axis_index("c")
    s_idx = lax.axis_index("s")
    wid = c_idx * NS + s_idx

    zf = jnp.zeros((L,), _f32)

    @pl.loop(0, B)
    def _zu(j):
        for k in range(D // L):
            rowsv0[j, pl.ds(k * L, L)] = zf

    @pl.loop(0, RPT // L)
    def _zd(j):
        zdv[pl.ds(j * L, L)] = zf

    @pl.loop(0, RPT // B)
    def _zs(i):
        pltpu.sync_copy(rowsv0, u_sh.at[pl.ds(s_idx * RPT + i * B, B)])

    pltpu.sync_copy(zdv, dn_sh.at[pl.ds(s_idx * RPT, RPT)])

    pltpu.sync_copy(as_hbm, asv)
    pltpu.sync_copy(c_hbm, cv)

    plsc.subcore_barrier()

    cvec = cv[...]
    ebase = wid * EPT
    NCH = EPT // B

    def _issue(c, b):
        base = ebase + c * B
        pltpu.sync_copy(src_hbm.at[pl.ds(base, B)], srcvs[b])
        pltpu.sync_copy(dst_hbm.at[pl.ds(base, B)], dstvs[b])
        pltpu.async_copy(h_hbm.at[srcvs[b]], rowsvs[b], gsem.at[b])
        pltpu.async_copy(ad_hbm.at[dstvs[b]], adgs[b], g2sem.at[b])

    def _wait_gather(b):
        pltpu.make_async_copy(h_hbm.at[srcvs[b]], rowsvs[b],
                              gsem.at[b]).wait()
        pltpu.make_async_copy(ad_hbm.at[dstvs[b]], adgs[b],
                              g2sem.at[b]).wait()

    def _wait_scatters(b):
        pltpu.make_async_copy(rowsvs[b], u_sh.at[dstvs[b]],
                              usem.at[b]).wait()
        pltpu.make_async_copy(pvs[b], dn_sh.at[dstvs[b]],
                              dsem.at[b]).wait()

    def _compute(c, b):
        for g in range(B // L):
            s16 = srcvs[b][pl.ds(g * L, L)]
            a = plsc.load_gather(asv, [s16]) + adgs[b][pl.ds(g * L, L)]
            e = jnp.where(a >= 0.0, a, 0.2 * a)
            p = jnp.exp(e - cvec)
            pvs[b][pl.ds(g * L, L)] = p

        @pl.loop(0, B)
        def _scale(j):
            pj = plsc.load_gather(pvs[b], [jnp.broadcast_to(j, (L,))])
            for k in range(D // L):
                sl = pl.ds(k * L, L)
                rowsvs[b][j, sl] = rowsvs[b][j, sl] * pj

        pltpu.async_copy(rowsvs[b], u_sh.at[dstvs[b]], usem.at[b],
                         add=True)
        pltpu.async_copy(pvs[b], dn_sh.at[dstvs[b]], dsem.at[b],
                         add=True)

    _issue(0, 0)

    @pl.loop(0, (NCH - 1) // NB)
    def _ring(k):
        for b in range(NB):
            c = k * NB + b
            nb = (b + 1) % NB
            _wait_gather(b)
            if b == NB - 1:
                _wait_scatters(nb)
            else:
                @pl.when(k >= 1)
                def _(nb=nb):
                    _wait_scatters(nb)
            _issue(c + 1, nb)
            _compute(c, b)

    _wait_gather(0)
    _wait_scatters(1)
    _issue(NCH - 1, 1)
    _compute(NCH - 2, 0)
    _wait_gather(1)
    _compute(NCH - 1, 1)
    for b in range(NB):
        _wait_scatters(b)

    plsc.subcore_barrier()

    r0 = s_idx * RPT
    pltpu.sync_copy(u_sh.at[pl.ds(r0, RPT)], u_hbm.at[c_idx, pl.ds(r0, RPT)])
    pltpu.sync_copy(dn_sh.at[pl.ds(r0, RPT)], dn_hbm.at[c_idx, pl.ds(r0, RPT)])


def _sc_compiler_params():
    cp = pltpu.CompilerParams()
    fields = pltpu.CompilerParams.__dataclass_fields__
    if "needs_layout_passes" in fields:
        cp = dataclasses.replace(cp, needs_layout_passes=False)
    if "use_tc_tiling_on_sc" in fields:
        cp = dataclasses.replace(cp, use_tc_tiling_on_sc=False)
    return cp


@functools.cache
def _edge_sc_kernel():
  return pl.kernel(
    _edge_body,
    compiler_params=_sc_compiler_params(),
    out_type=(jax.ShapeDtypeStruct((NC, NPAD, D), _f32),
              jax.ShapeDtypeStruct((NC, NPAD), _f32)),
    mesh=plsc.VectorSubcoreMesh(core_axis_name="c", subcore_axis_name="s",
                                num_cores=NC, num_subcores=NS),
    scratch_types=[
        pltpu.VMEM((N,), _f32),
        pltpu.VMEM((L,), _f32),
        pltpu.VMEM((B,), jnp.int32),
        pltpu.VMEM((B,), jnp.int32),
        pltpu.VMEM((B,), jnp.int32),
        pltpu.VMEM((B,), jnp.int32),
        pltpu.VMEM((B,), jnp.int32),
        pltpu.VMEM((B,), jnp.int32),
        pltpu.VMEM((B, D), _f32),
        pltpu.VMEM((B, D), _f32),
        pltpu.VMEM((B, D), _f32),
        pltpu.VMEM((B,), _f32),
        pltpu.VMEM((B,), _f32),
        pltpu.VMEM((B,), _f32),
        pltpu.VMEM((B,), _f32),
        pltpu.VMEM((B,), _f32),
        pltpu.VMEM((B,), _f32),
        pltpu.VMEM((RPT,), _f32),
        pltpu.VMEM_SHARED((NPAD, D), _f32),
        pltpu.VMEM_SHARED((NPAD,), _f32),
        pltpu.SemaphoreType.DMA((NB,)),
        pltpu.SemaphoreType.DMA((NB,)),
        pltpu.SemaphoreType.DMA((NB,)),
        pltpu.SemaphoreType.DMA((NB,)),
    ])


def _layer_edges(h, a_s, a_d, src, dst, c):
    c16 = jnp.broadcast_to(jnp.reshape(c, ()), (16,))
    u, dn = _edge_sc_kernel()(h, a_s.reshape(N), a_d.reshape(N), src, dst, c16)
    return (u[0, :N], u[1, :N],
            dn[0, :N].reshape(N, 1), dn[1, :N].reshape(N, 1))


def kernel(x, edge_index, W1, att_src1, att_dst1, bias1, gamma1, beta1,
           W2, att_src2, att_dst2, bias2):
    src = edge_index[0]
    dst = edge_index[1]
    h1, as1, ad1, c1 = _mat1(x, W1, att_src1.reshape(1, D),
                             att_dst1.reshape(1, D))
    u0, u1, d0, d1 = _layer_edges(h1, as1, ad1, src, dst, c1)
    h2, as2, ad2, c2 = _combine_mat(u0, u1, d0, d1, bias1.reshape(1, D),
                                    gamma1.reshape(1, D), beta1.reshape(1, D),
                                    W2, att_src2.reshape(1, D),
                                    att_dst2.reshape(1, D))
    u0, u1, d0, d1 = _layer_edges(h2, as2, ad2, src, dst, c2)
    return _final(u0, u1, d0, d1, bias2.reshape(1, D))


# speedup vs baseline: 37.6992x; 1.1749x over previous
import dataclasses
import functools

import jax
import jax.numpy as jnp
from jax import lax
from jax.experimental import pallas as pl
from jax.experimental.pallas import tpu as pltpu
from jax.experimental.pallas import tpu_sc as plsc

N = 10000
E = 320000
D = 128
NC = 2
NS = 16
L = 16
EPT = E // (NC * NS)
B = 80
NB = 3
NPAD = 10240
RPT = NPAD // NS

_f32 = jnp.float32


def _proj_tail(h, asv_ref, adv_ref, h_ref, as_ref, ad_ref, c_ref):
    h_ref[...] = h
    a_s = jnp.sum(h * asv_ref[...], axis=1, keepdims=True)
    a_d = jnp.sum(h * adv_ref[...], axis=1, keepdims=True)
    as_ref[...] = a_s
    ad_ref[...] = a_d
    m = jnp.max(a_s) + jnp.max(a_d)
    c_ref[0, 0] = jnp.where(m >= 0.0, m, 0.2 * m)


def _mat1_body(x_ref, w_ref, asv_ref, adv_ref, h_ref, as_ref, ad_ref, c_ref):
    h = jnp.dot(x_ref[...], w_ref[...], precision=lax.Precision.HIGHEST,
                preferred_element_type=_f32)
    _proj_tail(h, asv_ref, adv_ref, h_ref, as_ref, ad_ref, c_ref)


def _combine_mat_body(u0_ref, u1_ref, d0_ref, d1_ref, bias_ref, gamma_ref,
                      beta_ref, w_ref, asv_ref, adv_ref,
                      h_ref, as_ref, ad_ref, c_ref):
    denom = d0_ref[...] + d1_ref[...] + 1e-16
    z = (u0_ref[...] + u1_ref[...]) / denom + bias_ref[...]
    mean = jnp.mean(z, axis=0, keepdims=True)
    var = jnp.mean((z - mean) ** 2, axis=0, keepdims=True)
    z = gamma_ref[...] * (z - mean) / jnp.sqrt(var + 1e-5) + beta_ref[...]
    z = jnp.maximum(z, 0.0)
    h = jnp.dot(z, w_ref[...], precision=lax.Precision.HIGHEST,
                preferred_element_type=_f32)
    _proj_tail(h, asv_ref, adv_ref, h_ref, as_ref, ad_ref, c_ref)


def _final_body(u0_ref, u1_ref, d0_ref, d1_ref, bias_ref, o_ref):
    denom = d0_ref[...] + d1_ref[...] + 1e-16
    z = (u0_ref[...] + u1_ref[...]) / denom + bias_ref[...]
    o_ref[...] = jnp.maximum(z, 0.0)


_proj_out_shape = (jax.ShapeDtypeStruct((N, D), _f32),
                   jax.ShapeDtypeStruct((N, 1), _f32),
                   jax.ShapeDtypeStruct((N, 1), _f32),
                   jax.ShapeDtypeStruct((1, 1), _f32))
_proj_out_specs = (pl.BlockSpec(memory_space=pltpu.VMEM),
                   pl.BlockSpec(memory_space=pltpu.VMEM),
                   pl.BlockSpec(memory_space=pltpu.VMEM),
                   pl.BlockSpec(memory_space=pltpu.SMEM))

_mat1 = pl.pallas_call(
    _mat1_body, out_shape=_proj_out_shape, out_specs=_proj_out_specs)

_combine_mat = pl.pallas_call(
    _combine_mat_body, out_shape=_proj_out_shape, out_specs=_proj_out_specs)

_final = pl.pallas_call(
    _final_body,
    out_shape=jax.ShapeDtypeStruct((N, D), _f32))


def _edge_body(h_hbm, as_hbm, ad_hbm, src_hbm, dst_hbm, c_hbm,
               u_hbm, dn_hbm,
               asv, cv,
               srcv0, srcv1, srcv2, dstv0, dstv1, dstv2,
               rowsv0, rowsv1, rowsv2, pv0, pv1, pv2,
               adg0, adg1, adg2, zdv,
               u_sh, dn_sh, gsem, g2sem, usem, dsem, isem, jsem):
    srcvs = (srcv0, srcv1, srcv2)
    dstvs = (dstv0, dstv1, dstv2)
    rowsvs = (rowsv0, rowsv1, rowsv2)
    pvs = (pv0, pv1, pv2)
    adgs = (adg0, adg1, adg2)
    c_idx = lax.axis_index("c")
    s_idx = lax.axis_index("s")
    wid = c_idx * NS + s_idx

    zf = jnp.zeros((L,), _f32)

    @pl.loop(0, B)
    def _zu(j):
        for k in range(D // L):
            rowsv0[j, pl.ds(k * L, L)] = zf

    @pl.loop(0, RPT // L)
    def _zd(j):
        zdv[pl.ds(j * L, L)] = zf

    @pl.loop(0, RPT // B)
    def _zs(i):
        pltpu.sync_copy(rowsv0, u_sh.at[pl.ds(s_idx * RPT + i * B, B)])

    pltpu.sync_copy(zdv, dn_sh.at[pl.ds(s_idx * RPT, RPT)])

    pltpu.sync_copy(as_hbm, asv)
    pltpu.sync_copy(c_hbm, cv)

    plsc.subcore_barrier()

    cvec = cv[...]
    ebase = wid * EPT
    NCH = EPT // B

    def _issue_idx(c, b):
        base = ebase + c * B
        pltpu.async_copy(src_hbm.at[pl.ds(base, B)], srcvs[b], isem.at[b])
        pltpu.async_copy(dst_hbm.at[pl.ds(base, B)], dstvs[b], jsem.at[b])

    def _issue_gathers(c, b):
        base = ebase + c * B
        pltpu.make_async_copy(src_hbm.at[pl.ds(base, B)], srcvs[b],
                              isem.at[b]).wait()
        pltpu.make_async_copy(dst_hbm.at[pl.ds(base, B)], dstvs[b],
                              jsem.at[b]).wait()
        pltpu.async_copy(h_hbm.at[srcvs[b]], rowsvs[b], gsem.at[b])
        pltpu.async_copy(ad_hbm.at[dstvs[b]], adgs[b], g2sem.at[b])

    def _wait_gather(b):
        pltpu.make_async_copy(h_hbm.at[srcvs[b]], rowsvs[b],
                              gsem.at[b]).wait()
        pltpu.make_async_copy(ad_hbm.at[dstvs[b]], adgs[b],
                              g2sem.at[b]).wait()

    def _wait_scatters(b):
        pltpu.make_async_copy(rowsvs[b], u_sh.at[dstvs[b]],
                              usem.at[b]).wait()
        pltpu.make_async_copy(pvs[b], dn_sh.at[dstvs[b]],
                              dsem.at[b]).wait()

    def _pcompute(b):
        for g in range(B // L):
            s16 = srcvs[b][pl.ds(g * L, L)]
            a = plsc.load_gather(asv, [s16]) + adgs[b][pl.ds(g * L, L)]
            e = jnp.where(a >= 0.0, a, 0.2 * a)
            p = jnp.exp(e - cvec)
            pvs[b][pl.ds(g * L, L)] = p

    def _finish(b):
        @pl.loop(0, B)
        def _scale(j):
            pj = plsc.load_gather(pvs[b], [jnp.broadcast_to(j, (L,))])
            for k in range(D // L):
                sl = pl.ds(k * L, L)
                rowsvs[b][j, sl] = rowsvs[b][j, sl] * pj

        pltpu.async_copy(rowsvs[b], u_sh.at[dstvs[b]], usem.at[b],
                         add=True)
        pltpu.async_copy(pvs[b], dn_sh.at[dstvs[b]], dsem.at[b],
                         add=True)

    _issue_idx(0, 0)
    _issue_gathers(0, 0)

    @pl.loop(0, (NCH - 1) // NB)
    def _ring(k):
        for b in range(NB):
            c = k * NB + b
            nb = (b + 1) % NB
            _wait_gather(b)
            if b == NB - 1:
                _wait_scatters(nb)
            else:
                @pl.when(k >= 1)
                def _(nb=nb):
                    _wait_scatters(nb)
            _issue_idx(c + 1, nb)
            _pcompute(b)
            _issue_gathers(c + 1, nb)
            _finish(b)

    _wait_gather(0)
    _wait_scatters(1)
    _issue_idx(NCH - 1, 1)
    _pcompute(0)
    _issue_gathers(NCH - 1, 1)
    _finish(0)
    _wait_gather(1)
    _pcompute(1)
    _finish(1)
    for b in range(NB):
        _wait_scatters(b)

    plsc.subcore_barrier()

    r0 = s_idx * RPT
    pltpu.sync_copy(u_sh.at[pl.ds(r0, RPT)], u_hbm.at[c_idx, pl.ds(r0, RPT)])
    pltpu.sync_copy(dn_sh.at[pl.ds(r0, RPT)], dn_hbm.at[c_idx, pl.ds(r0, RPT)])


def _sc_compiler_params():
    cp = pltpu.CompilerParams()
    fields = pltpu.CompilerParams.__dataclass_fields__
    if "needs_layout_passes" in fields:
        cp = dataclasses.replace(cp, needs_layout_passes=False)
    if "use_tc_tiling_on_sc" in fields:
        cp = dataclasses.replace(cp, use_tc_tiling_on_sc=False)
    return cp


@functools.cache
def _edge_sc_kernel():
  return pl.kernel(
    _edge_body,
    compiler_params=_sc_compiler_params(),
    out_type=(jax.ShapeDtypeStruct((NC, NPAD, D), _f32),
              jax.ShapeDtypeStruct((NC, NPAD), _f32)),
    mesh=plsc.VectorSubcoreMesh(core_axis_name="c", subcore_axis_name="s",
                                num_cores=NC, num_subcores=NS),
    scratch_types=[
        pltpu.VMEM((N,), _f32),
        pltpu.VMEM((L,), _f32),
        pltpu.VMEM((B,), jnp.int32),
        pltpu.VMEM((B,), jnp.int32),
        pltpu.VMEM((B,), jnp.int32),
        pltpu.VMEM((B,), jnp.int32),
        pltpu.VMEM((B,), jnp.int32),
        pltpu.VMEM((B,), jnp.int32),
        pltpu.VMEM((B, D), _f32),
        pltpu.VMEM((B, D), _f32),
        pltpu.VMEM((B, D), _f32),
        pltpu.VMEM((B,), _f32),
        pltpu.VMEM((B,), _f32),
        pltpu.VMEM((B,), _f32),
        pltpu.VMEM((B,), _f32),
        pltpu.VMEM((B,), _f32),
        pltpu.VMEM((B,), _f32),
        pltpu.VMEM((RPT,), _f32),
        pltpu.VMEM_SHARED((NPAD, D), _f32),
        pltpu.VMEM_SHARED((NPAD,), _f32),
        pltpu.SemaphoreType.DMA((NB,)),
        pltpu.SemaphoreType.DMA((NB,)),
        pltpu.SemaphoreType.DMA((NB,)),
        pltpu.SemaphoreType.DMA((NB,)),
        pltpu.SemaphoreType.DMA((NB,)),
        pltpu.SemaphoreType.DMA((NB,)),
    ])


def _layer_edges(h, a_s, a_d, src, dst, c):
    c16 = jnp.broadcast_to(jnp.reshape(c, ()), (16,))
    u, dn = _edge_sc_kernel()(h, a_s.reshape(N), a_d.reshape(N), src, dst, c16)
    return (u[0, :N], u[1, :N],
            dn[0, :N].reshape(N, 1), dn[1, :N].reshape(N, 1))


def kernel(x, edge_index, W1, att_src1, att_dst1, bias1, gamma1, beta1,
           W2, att_src2, att_dst2, bias2):
    src = edge_index[0]
    dst = edge_index[1]
    h1, as1, ad1, c1 = _mat1(x, W1, att_src1.reshape(1, D),
                             att_dst1.reshape(1, D))
    u0, u1, d0, d1 = _layer_edges(h1, as1, ad1, src, dst, c1)
    h2, as2, ad2, c2 = _combine_mat(u0, u1, d0, d1, bias1.reshape(1, D),
                                    gamma1.reshape(1, D), beta1.reshape(1, D),
                                    W2, att_src2.reshape(1, D),
                                    att_dst2.reshape(1, D))
    u0, u1, d0, d1 = _layer_edges(h2, as2, ad2, src, dst, c2)
    return _final(u0, u1, d0, d1, bias2.reshape(1, D))


# speedup vs baseline: 37.9161x; 1.0058x over previous
import dataclasses
import functools

import jax
import jax.numpy as jnp
from jax import lax
from jax.experimental import pallas as pl
from jax.experimental.pallas import tpu as pltpu
from jax.experimental.pallas import tpu_sc as plsc

N = 10000
E = 320000
D = 128
NC = 2
NS = 16
L = 16
EPT = E // (NC * NS)
B = 80
NB = 3
NPAD = 10240
RPT = NPAD // NS

_f32 = jnp.float32


def _proj_tail(h, asv_ref, adv_ref, h_ref, as_ref, ad_ref, c_ref):
    h_ref[...] = h
    a_s = jnp.sum(h * asv_ref[...], axis=1, keepdims=True)
    a_d = jnp.sum(h * adv_ref[...], axis=1, keepdims=True)
    as_ref[...] = a_s
    ad_ref[...] = a_d
    m = jnp.max(a_s) + jnp.max(a_d)
    c_ref[0, 0] = jnp.where(m >= 0.0, m, 0.2 * m)


def _mat1_body(x_ref, w_ref, asv_ref, adv_ref, h_ref, as_ref, ad_ref, c_ref):
    h = jnp.dot(x_ref[...], w_ref[...], precision=lax.Precision.HIGHEST,
                preferred_element_type=_f32)
    _proj_tail(h, asv_ref, adv_ref, h_ref, as_ref, ad_ref, c_ref)


def _combine_mat_body(u0_ref, u1_ref, d0_ref, d1_ref, bias_ref, gamma_ref,
                      beta_ref, w_ref, asv_ref, adv_ref,
                      h_ref, as_ref, ad_ref, c_ref):
    denom = d0_ref[...] + d1_ref[...] + 1e-16
    z = (u0_ref[...] + u1_ref[...]) / denom + bias_ref[...]
    mean = jnp.mean(z, axis=0, keepdims=True)
    var = jnp.mean((z - mean) ** 2, axis=0, keepdims=True)
    z = gamma_ref[...] * (z - mean) / jnp.sqrt(var + 1e-5) + beta_ref[...]
    z = jnp.maximum(z, 0.0)
    h = jnp.dot(z, w_ref[...], precision=lax.Precision.HIGHEST,
                preferred_element_type=_f32)
    _proj_tail(h, asv_ref, adv_ref, h_ref, as_ref, ad_ref, c_ref)


def _final_body(u0_ref, u1_ref, d0_ref, d1_ref, bias_ref, o_ref):
    denom = d0_ref[...] + d1_ref[...] + 1e-16
    z = (u0_ref[...] + u1_ref[...]) / denom + bias_ref[...]
    o_ref[...] = jnp.maximum(z, 0.0)


_proj_out_shape = (jax.ShapeDtypeStruct((N, D), _f32),
                   jax.ShapeDtypeStruct((N, 1), _f32),
                   jax.ShapeDtypeStruct((N, 1), _f32),
                   jax.ShapeDtypeStruct((1, 1), _f32))
_proj_out_specs = (pl.BlockSpec(memory_space=pltpu.VMEM),
                   pl.BlockSpec(memory_space=pltpu.VMEM),
                   pl.BlockSpec(memory_space=pltpu.VMEM),
                   pl.BlockSpec(memory_space=pltpu.SMEM))

_mat1 = pl.pallas_call(
    _mat1_body, out_shape=_proj_out_shape, out_specs=_proj_out_specs)

_combine_mat = pl.pallas_call(
    _combine_mat_body, out_shape=_proj_out_shape, out_specs=_proj_out_specs)

_final = pl.pallas_call(
    _final_body,
    out_shape=jax.ShapeDtypeStruct((N, D), _f32))


def _edge_body(h_hbm, as_hbm, ad_hbm, src_hbm, dst_hbm, c_hbm,
               u_hbm, dn_hbm,
               asv, cv,
               srcv0, srcv1, srcv2, dstv0, dstv1, dstv2,
               rowsv0, rowsv1, rowsv2, pv0, pv1, pv2,
               adg0, adg1, adg2, zdv,
               u_sh, dn_sh, gsem, g2sem, usem, dsem, isem, jsem):
    srcvs = (srcv0, srcv1, srcv2)
    dstvs = (dstv0, dstv1, dstv2)
    rowsvs = (rowsv0, rowsv1, rowsv2)
    pvs = (pv0, pv1, pv2)
    adgs = (adg0, adg1, adg2)
    c_idx = lax.axis_index("c")
    s_idx = lax.axis_index("s")
    wid = c_idx * NS + s_idx

    zf = jnp.zeros((L,), _f32)

    @pl.loop(0, B)
    def _zu(j):
        for k in range(D // L):
            rowsv0[j, pl.ds(k * L, L)] = zf

    @pl.loop(0, RPT // L)
    def _zd(j):
        zdv[pl.ds(j * L, L)] = zf

    @pl.loop(0, RPT // B)
    def _zs(i):
        pltpu.sync_copy(rowsv0, u_sh.at[pl.ds(s_idx * RPT + i * B, B)])

    pltpu.sync_copy(zdv, dn_sh.at[pl.ds(s_idx * RPT, RPT)])

    pltpu.sync_copy(as_hbm, asv)
    pltpu.sync_copy(c_hbm, cv)

    plsc.subcore_barrier()

    cvec = cv[...]
    ebase = wid * EPT
    NCH = EPT // B

    def _issue_idx(c, b):
        base = ebase + c * B
        pltpu.async_copy(src_hbm.at[pl.ds(base, B)], srcvs[b], isem.at[b])
        pltpu.async_copy(dst_hbm.at[pl.ds(base, B)], dstvs[b], jsem.at[b])

    def _issue_gathers(c, b):
        base = ebase + c * B
        pltpu.make_async_copy(src_hbm.at[pl.ds(base, B)], srcvs[b],
                              isem.at[b]).wait()
        pltpu.make_async_copy(dst_hbm.at[pl.ds(base, B)], dstvs[b],
                              jsem.at[b]).wait()
        pltpu.async_copy(h_hbm.at[srcvs[b]], rowsvs[b], gsem.at[b])
        pltpu.async_copy(ad_hbm.at[dstvs[b]], adgs[b], g2sem.at[b])

    def _wait_gather(b):
        pltpu.make_async_copy(h_hbm.at[srcvs[b]], rowsvs[b],
                              gsem.at[b]).wait()
        pltpu.make_async_copy(ad_hbm.at[dstvs[b]], adgs[b],
                              g2sem.at[b]).wait()

    def _wait_scatters(b):
        pltpu.make_async_copy(rowsvs[b], u_sh.at[dstvs[b]],
                              usem.at[b]).wait()
        pltpu.make_async_copy(pvs[b], dn_sh.at[dstvs[b]],
                              dsem.at[b]).wait()

    def _pcompute(b):
        for g in range(B // L):
            s16 = srcvs[b][pl.ds(g * L, L)]
            a = plsc.load_gather(asv, [s16]) + adgs[b][pl.ds(g * L, L)]
            e = jnp.where(a >= 0.0, a, 0.2 * a)
            p = jnp.exp(e - cvec)
            pvs[b][pl.ds(g * L, L)] = p

    def _finish(b):
        @plsc.parallel_loop(0, B, unroll=2)
        def _scale(j):
            pj = plsc.load_gather(pvs[b], [jnp.broadcast_to(j, (L,))])
            for k in range(D // L):
                sl = pl.ds(k * L, L)
                rowsvs[b][j, sl] = rowsvs[b][j, sl] * pj

        pltpu.async_copy(rowsvs[b], u_sh.at[dstvs[b]], usem.at[b],
                         add=True)
        pltpu.async_copy(pvs[b], dn_sh.at[dstvs[b]], dsem.at[b],
                         add=True)

    _issue_idx(0, 0)
    _issue_gathers(0, 0)

    @pl.loop(0, (NCH - 1) // NB)
    def _ring(k):
        for b in range(NB):
            c = k * NB + b
            nb = (b + 1) % NB
            _wait_gather(b)
            if b == NB - 1:
                _wait_scatters(nb)
            else:
                @pl.when(k >= 1)
                def _(nb=nb):
                    _wait_scatters(nb)
            _issue_idx(c + 1, nb)
            _pcompute(b)
            _issue_gathers(c + 1, nb)
            _finish(b)

    _wait_gather(0)
    _wait_scatters(1)
    _issue_idx(NCH - 1, 1)
    _pcompute(0)
    _issue_gathers(NCH - 1, 1)
    _finish(0)
    _wait_gather(1)
    _pcompute(1)
    _finish(1)
    for b in range(NB):
        _wait_scatters(b)

    plsc.subcore_barrier()

    r0 = s_idx * RPT
    pltpu.sync_copy(u_sh.at[pl.ds(r0, RPT)], u_hbm.at[c_idx, pl.ds(r0, RPT)])
    pltpu.sync_copy(dn_sh.at[pl.ds(r0, RPT)], dn_hbm.at[c_idx, pl.ds(r0, RPT)])


def _sc_compiler_params():
    cp = pltpu.CompilerParams()
    fields = pltpu.CompilerParams.__dataclass_fields__
    if "needs_layout_passes" in fields:
        cp = dataclasses.replace(cp, needs_layout_passes=False)
    if "use_tc_tiling_on_sc" in fields:
        cp = dataclasses.replace(cp, use_tc_tiling_on_sc=False)
    return cp


@functools.cache
def _edge_sc_kernel():
  return pl.kernel(
    _edge_body,
    compiler_params=_sc_compiler_params(),
    out_type=(jax.ShapeDtypeStruct((NC, NPAD, D), _f32),
              jax.ShapeDtypeStruct((NC, NPAD), _f32)),
    mesh=plsc.VectorSubcoreMesh(core_axis_name="c", subcore_axis_name="s",
                                num_cores=NC, num_subcores=NS),
    scratch_types=[
        pltpu.VMEM((N,), _f32),
        pltpu.VMEM((L,), _f32),
        pltpu.VMEM((B,), jnp.int32),
        pltpu.VMEM((B,), jnp.int32),
        pltpu.VMEM((B,), jnp.int32),
        pltpu.VMEM((B,), jnp.int32),
        pltpu.VMEM((B,), jnp.int32),
        pltpu.VMEM((B,), jnp.int32),
        pltpu.VMEM((B, D), _f32),
        pltpu.VMEM((B, D), _f32),
        pltpu.VMEM((B, D), _f32),
        pltpu.VMEM((B,), _f32),
        pltpu.VMEM((B,), _f32),
        pltpu.VMEM((B,), _f32),
        pltpu.VMEM((B,), _f32),
        pltpu.VMEM((B,), _f32),
        pltpu.VMEM((B,), _f32),
        pltpu.VMEM((RPT,), _f32),
        pltpu.VMEM_SHARED((NPAD, D), _f32),
        pltpu.VMEM_SHARED((NPAD,), _f32),
        pltpu.SemaphoreType.DMA((NB,)),
        pltpu.SemaphoreType.DMA((NB,)),
        pltpu.SemaphoreType.DMA((NB,)),
        pltpu.SemaphoreType.DMA((NB,)),
        pltpu.SemaphoreType.DMA((NB,)),
        pltpu.SemaphoreType.DMA((NB,)),
    ])


def _layer_edges(h, a_s, a_d, src, dst, c):
    c16 = jnp.broadcast_to(jnp.reshape(c, ()), (16,))
    u, dn = _edge_sc_kernel()(h, a_s.reshape(N), a_d.reshape(N), src, dst, c16)
    return (u[0, :N], u[1, :N],
            dn[0, :N].reshape(N, 1), dn[1, :N].reshape(N, 1))


def kernel(x, edge_index, W1, att_src1, att_dst1, bias1, gamma1, beta1,
           W2, att_src2, att_dst2, bias2):
    src = edge_index[0]
    dst = edge_index[1]
    h1, as1, ad1, c1 = _mat1(x, W1, att_src1.reshape(1, D),
                             att_dst1.reshape(1, D))
    u0, u1, d0, d1 = _layer_edges(h1, as1, ad1, src, dst, c1)
    h2, as2, ad2, c2 = _combine_mat(u0, u1, d0, d1, bias1.reshape(1, D),
                                    gamma1.reshape(1, D), beta1.reshape(1, D),
                                    W2, att_src2.reshape(1, D),
                                    att_dst2.reshape(1, D))
    u0, u1, d0, d1 = _layer_edges(h2, as2, ad2, src, dst, c2)
    return _final(u0, u1, d0, d1, bias2.reshape(1, D))
